# Initial kernel scaffold; baseline (speedup 1.0000x reference)
import jax
import jax.numpy as jnp
from jax.experimental import pallas as pl


def kernel(x, edge_index, W1, b1, W_mu, b_mu, W_ls, b_ls):
    raise NotImplementedError("write your pallas kernel here")


# speedup vs baseline: 55.8552x; 55.8552x over previous
import functools

import jax
import jax.numpy as jnp
from jax import lax
from jax.experimental import pallas as pl
from jax.experimental.pallas import tpu as pltpu
from jax.experimental.pallas import tpu_sc as plsc

NC = 2
NS = 16
NW = NC * NS
CH = 80
U = 10
UA = 5

_f32 = jnp.float32


def _round_up(v, m):
    return (v + m - 1) // m * m


def _make_degree_kernel(n_pad, e_pad):
    per_w = e_pad // NW
    n_ch = per_w // CH
    rows_pt = n_pad // NS
    mesh = plsc.VectorSubcoreMesh(core_axis_name="c", subcore_axis_name="s")
    n_wave = n_ch // U
    tail = n_ch - n_wave * U

    @functools.partial(
        pl.kernel,
        out_type=jax.ShapeDtypeStruct((NC, 1, n_pad), _f32),
        mesh=mesh,
        compiler_params=pltpu.CompilerParams(use_tc_tiling_on_sc=False),
        scratch_types=[
            pltpu.VMEM((n_ch, CH), jnp.int32),
            pltpu.VMEM((CH,), _f32),
            pltpu.VMEM((rows_pt,), _f32),
            pltpu.VMEM_SHARED((n_pad,), _f32),
            pltpu.SemaphoreType.DMA,
        ],
    )
    def deg_kernel(dst_hbm, out_hbm, didx2, ones_v, stage, acc, ssem):
        cid = lax.axis_index("c")
        sid = lax.axis_index("s")
        wid = sid * NC + cid

        pltpu.sync_copy(dst_hbm.at[pl.ds(wid * n_ch, n_ch)], didx2)

        def _fill(i, _):
            ones_v[pl.ds(i * 16, 16)] = jnp.ones((16,), _f32)
            return 0
        lax.fori_loop(0, CH // 16, _fill, 0)

        def _zero(i, _):
            stage[pl.ds(i * 16, 16)] = jnp.zeros((16,), _f32)
            return 0
        lax.fori_loop(0, rows_pt // 16, _zero, 0)
        pltpu.sync_copy(stage, acc.at[pl.ds(sid * rows_pt, rows_pt)])
        plsc.subcore_barrier()

        def _wave(wv, _):
            base = wv * U
            descs = [pltpu.async_copy(ones_v, acc.at[didx2.at[base + b]],
                                      ssem, add=True)
                     for b in range(U)]
            for d in descs:
                d.wait()
            return 0
        lax.fori_loop(0, n_wave, _wave, 0)
        if tail:
            descs = [pltpu.async_copy(ones_v,
                                      acc.at[didx2.at[n_wave * U + b]],
                                      ssem, add=True)
                     for b in range(tail)]
            for d in descs:
                d.wait()
        plsc.subcore_barrier()

        pltpu.sync_copy(acc.at[pl.ds(sid * rows_pt, rows_pt)], stage)
        pltpu.sync_copy(stage,
                        out_hbm.at[cid, 0, pl.ds(sid * rows_pt, rows_pt)])

    return deg_kernel


def _make_mega_kernel(n_pad, e_pad, width):
    half = width // NC
    per_t = e_pad // NS
    n_ch = per_t // CH
    rows_pt = n_pad // NS
    n_wave = n_ch // UA
    tail = n_ch - n_wave * UA
    n_pair = (n_wave - 1) // 2
    rem = n_wave - 2 * n_pair
    mesh = plsc.VectorSubcoreMesh(core_axis_name="c", subcore_axis_name="s")

    @functools.partial(
        pl.kernel,
        out_type=(
            jax.ShapeDtypeStruct((NC, n_pad, half), _f32),
            jax.ShapeDtypeStruct((NC * n_pad, half), _f32),
        ),
        mesh=mesh,
        compiler_params=pltpu.CompilerParams(use_tc_tiling_on_sc=False),
        scratch_types=[
            pltpu.VMEM((n_ch, CH), jnp.int32),
            pltpu.VMEM((n_ch, CH), jnp.int32),
            pltpu.VMEM((2, UA, CH, half), _f32),
            pltpu.VMEM((rows_pt, half), _f32),
            pltpu.VMEM((rows_pt, half), _f32),
            pltpu.VMEM((rows_pt, half), _f32),
            pltpu.VMEM((rows_pt, half), _f32),
            pltpu.VMEM((rows_pt, half), _f32),
            pltpu.VMEM((rows_pt, half), _f32),
            pltpu.VMEM((16,), _f32),
            pltpu.VMEM_SHARED((n_pad, half), _f32),
            pltpu.SemaphoreType.DMA,
            pltpu.SemaphoreType.DMA,
            pltpu.SemaphoreType.DMA,
        ],
    )
    def mega_kernel(g1_hbm, sp_hbm, disb_hbm, invb_hbm, b1_hbm,
                    src_hbm, dst_hbm, out_hbm, g2_hbm,
                    sidx2, didx2, rows, stage, invh, sbuf, spl,
                    disb, invb, bloc, acc1, gsem0, gsem1, ssem):
        cid = lax.axis_index("c")
        sid = lax.axis_index("s")
        r0 = sid * rows_pt
        goff = cid * n_pad
        gsems = (gsem0, gsem1)

        pltpu.sync_copy(src_hbm.at[pl.ds(sid * n_ch, n_ch)], sidx2)
        pltpu.sync_copy(dst_hbm.at[pl.ds(sid * n_ch, n_ch)], didx2)
        pltpu.sync_copy(disb_hbm.at[pl.ds(r0, rows_pt)], disb)
        pltpu.sync_copy(invb_hbm.at[pl.ds(r0, rows_pt)], invb)
        pltpu.sync_copy(sp_hbm.at[pl.ds(goff + r0, rows_pt)], spl)
        pltpu.sync_copy(b1_hbm.at[pl.ds(cid * half, half)], bloc)

        def _off(i, _):
            for j in range(CH // 16):
                sl = pl.ds(j * 16, 16)
                sidx2[i, sl] = sidx2[i, sl] + goff
            return 0
        lax.fori_loop(0, n_ch, _off, 0)

        def _zero(i, _):
            stage[i, pl.ds(0, 16)] = jnp.zeros((16,), _f32)
            return 0
        lax.fori_loop(0, rows_pt, _zero, 0)
        pltpu.sync_copy(stage, acc1.at[pl.ds(r0, rows_pt)])
        plsc.subcore_barrier()

        def _run_agg(g_src, acc):
            def _fire_g(wv, bank, count=UA):
                return [pltpu.async_copy(g_src.at[sidx2.at[wv * UA + b]],
                                         rows.at[bank, b], gsems[bank])
                        for b in range(count)]

            def _drain_consume(wv, bank, count=UA):
                for b in range(count):
                    pltpu.make_async_copy(g_src.at[sidx2.at[wv * UA + b]],
                                          rows.at[bank, b],
                                          gsems[bank]).wait()
                sds = [pltpu.async_copy(rows.at[bank, b],
                                        acc.at[didx2.at[wv * UA + b]],
                                        ssem, add=True)
                       for b in range(count)]
                for d in sds:
                    d.wait()

            _fire_g(0, 0)

            def _pair(i, _):
                wa = 2 * i
                _fire_g(wa + 1, 1)
                _drain_consume(wa, 0)
                _fire_g(wa + 2, 0)
                _drain_consume(wa + 1, 1)
                return 0
            lax.fori_loop(0, n_pair, _pair, 0)

            if rem == 2:
                wa = 2 * n_pair
                _fire_g(wa + 1, 1)
                _drain_consume(wa, 0)
                _drain_consume(wa + 1, 1)
            else:
                _drain_consume(2 * n_pair, 0)
            if tail:
                base = n_wave * UA
                gds = [pltpu.async_copy(g_src.at[sidx2.at[base + b]],
                                        rows.at[0, b], gsem0)
                       for b in range(tail)]
                for d in gds:
                    d.wait()
                sds = [pltpu.async_copy(rows.at[0, b],
                                        acc.at[didx2.at[base + b]],
                                        ssem, add=True)
                       for b in range(tail)]
                for d in sds:
                    d.wait()

        _run_agg(g1_hbm, acc1)
        plsc.subcore_barrier()

        pltpu.sync_copy(acc1.at[pl.ds(r0, rows_pt)], sbuf)
        bv = bloc[pl.ds(0, 16)]

        def _row(r, _):
            sl = pl.ds(0, 16)
            dv = disb[r, sl]
            h = jnp.maximum(sbuf[r, sl] * dv + spl[r, sl] + bv, 0.0)
            stage[r, sl] = h * dv
            invh[r, sl] = h * invb[r, sl]
            return 0
        lax.fori_loop(0, rows_pt, _row, 0)
        pltpu.sync_copy(stage, g2_hbm.at[pl.ds(goff + r0, rows_pt)])
        def _zero2(i, _):
            stage[i, pl.ds(0, 16)] = jnp.zeros((16,), _f32)
            return 0
        lax.fori_loop(0, rows_pt, _zero2, 0)
        pltpu.sync_copy(stage, acc1.at[pl.ds(r0, rows_pt)])
        plsc.subcore_barrier()

        _run_agg(g2_hbm, acc1)
        plsc.subcore_barrier()

        pltpu.sync_copy(acc1.at[pl.ds(r0, rows_pt)], sbuf)

        def _row2(r, _):
            sl = pl.ds(0, 16)
            stage[r, sl] = sbuf[r, sl] * disb[r, sl] + invh[r, sl]
            return 0
        lax.fori_loop(0, rows_pt, _row2, 0)
        pltpu.sync_copy(stage, out_hbm.at[cid, pl.ds(r0, rows_pt)])

    return mega_kernel


def _prep(x, w1, ccol, n_pad):
    n = x.shape[0]
    hid = w1.shape[1]
    half = hid // NC

    def body(x_ref, w_ref, c_ref, g_ref, sp_ref, disb_ref, invb_ref):
        c = c_ref[...]
        dis = lax.rsqrt(c)
        inv = 1.0 / c
        p = jnp.dot(x_ref[...], w_ref[...], preferred_element_type=_f32)
        pd = p * dis[:n]
        pi = p * inv[:n]
        zpad = jnp.zeros((n_pad - n, half), _f32)
        ones_h = jnp.ones((1, half), _f32)
        disb_ref[...] = dis * ones_h
        invb_ref[...] = inv * ones_h
        g_ref[:n, :] = pd[:, :half]
        g_ref[n:n_pad, :] = zpad
        g_ref[n_pad:n_pad + n, :] = pd[:, half:]
        g_ref[n_pad + n:, :] = zpad
        sp_ref[:n, :] = pi[:, :half]
        sp_ref[n:n_pad, :] = zpad
        sp_ref[n_pad:n_pad + n, :] = pi[:, half:]
        sp_ref[n_pad + n:, :] = zpad

    return pl.pallas_call(
        body,
        out_shape=(
            jax.ShapeDtypeStruct((NC * n_pad, half), _f32),
            jax.ShapeDtypeStruct((NC * n_pad, half), _f32),
            jax.ShapeDtypeStruct((n_pad, half), _f32),
            jax.ShapeDtypeStruct((n_pad, half), _f32),
        ),
    )(x, w1, ccol)


def _fin(s, wmu, bmu, wls, bls, n):
    half = s.shape[2]

    def body(s_ref, wmu_ref, bmu_ref, wls_ref, bls_ref, mu_ref, ls_ref):
        a0 = s_ref[0, :n, :]
        a1 = s_ref[1, :n, :]
        wmu = wmu_ref[...]
        wls = wls_ref[...]
        mu_ref[...] = (jnp.dot(a0, wmu[:half, :], preferred_element_type=_f32)
                       + jnp.dot(a1, wmu[half:, :],
                                 preferred_element_type=_f32)
                       + bmu_ref[...])
        ls_ref[...] = (jnp.dot(a0, wls[:half, :], preferred_element_type=_f32)
                       + jnp.dot(a1, wls[half:, :],
                                 preferred_element_type=_f32)
                       + bls_ref[...])

    out_d = wmu.shape[1]
    return pl.pallas_call(
        body,
        out_shape=(
            jax.ShapeDtypeStruct((n, out_d), _f32),
            jax.ShapeDtypeStruct((n, out_d), _f32),
        ),
    )(s, wmu, bmu, wls, bls)


def kernel(x, edge_index, W1, b1, W_mu, b_mu, W_ls, b_ls):
    n, _ = x.shape
    hid = W1.shape[1]
    out_d = W_mu.shape[1]
    e = edge_index.shape[1]

    n_pad = _round_up(n + 1, NS * 128)
    e_pad = _round_up(e, NW * CH)

    src = edge_index[0]
    dst = edge_index[1]
    if e_pad != e:
        pad = e_pad - e
        src = jnp.concatenate([src, jnp.zeros((pad,), jnp.int32)])
        dst = jnp.concatenate([dst, jnp.full((pad,), n, jnp.int32)])
    src = src.reshape(e_pad // CH, CH)
    dst = dst.reshape(e_pad // CH, CH)

    deg_k = _make_degree_kernel(n_pad, e_pad)
    mega_k = _make_mega_kernel(n_pad, e_pad, hid)

    cnt = deg_k(dst)
    ccol = (cnt[0, 0] + cnt[1, 0] + 1.0).reshape(n_pad, 1)

    g1, sp1, disb, invb = _prep(x, W1, ccol, n_pad)
    out_agg, _g2 = mega_k(g1, sp1, disb, invb, b1, src, dst)
    mu, ls = _fin(out_agg, W_mu, b_mu.reshape(1, out_d),
                  W_ls, b_ls.reshape(1, out_d), n)
    return (mu, ls)


# speedup vs baseline: 69.2906x; 1.2405x over previous
import functools

import jax
import jax.numpy as jnp
from jax import lax
from jax.experimental import pallas as pl
from jax.experimental.pallas import tpu as pltpu
from jax.experimental.pallas import tpu_sc as plsc

NC = 2
NS = 16
NW = NC * NS
CH = 80
U = 10
UA = 5

_f32 = jnp.float32


def _round_up(v, m):
    return (v + m - 1) // m * m


def _make_degree_kernel(n_pad, e_pad):
    per_w = e_pad // NW
    n_ch = per_w // CH
    rows_pt = n_pad // NS
    mesh = plsc.VectorSubcoreMesh(core_axis_name="c", subcore_axis_name="s")
    n_wave = n_ch // U
    tail = n_ch - n_wave * U

    @functools.partial(
        pl.kernel,
        out_type=jax.ShapeDtypeStruct((NC, 1, n_pad), _f32),
        mesh=mesh,
        compiler_params=pltpu.CompilerParams(use_tc_tiling_on_sc=False),
        scratch_types=[
            pltpu.VMEM((n_ch, CH), jnp.int32),
            pltpu.VMEM((CH,), _f32),
            pltpu.VMEM((rows_pt,), _f32),
            pltpu.VMEM_SHARED((n_pad,), _f32),
            pltpu.SemaphoreType.DMA,
        ],
    )
    def deg_kernel(ei_hbm, out_hbm, didx2, ones_v, stage, acc, ssem):
        cid = lax.axis_index("c")
        sid = lax.axis_index("s")
        wid = sid * NC + cid

        pltpu.sync_copy(ei_hbm.at[1, pl.ds(wid * n_ch, n_ch)], didx2)

        def _fill(i, _):
            ones_v[pl.ds(i * 16, 16)] = jnp.ones((16,), _f32)
            return 0
        lax.fori_loop(0, CH // 16, _fill, 0)

        def _zero(i, _):
            stage[pl.ds(i * 16, 16)] = jnp.zeros((16,), _f32)
            return 0
        lax.fori_loop(0, rows_pt // 16, _zero, 0)
        pltpu.sync_copy(stage, acc.at[pl.ds(sid * rows_pt, rows_pt)])
        plsc.subcore_barrier()

        def _wave(wv, _):
            base = wv * U
            descs = [pltpu.async_copy(ones_v, acc.at[didx2.at[base + b]],
                                      ssem, add=True)
                     for b in range(U)]
            for d in descs:
                d.wait()
            return 0
        lax.fori_loop(0, n_wave, _wave, 0)
        if tail:
            descs = [pltpu.async_copy(ones_v,
                                      acc.at[didx2.at[n_wave * U + b]],
                                      ssem, add=True)
                     for b in range(tail)]
            for d in descs:
                d.wait()
        plsc.subcore_barrier()

        pltpu.sync_copy(acc.at[pl.ds(sid * rows_pt, rows_pt)], stage)
        pltpu.sync_copy(stage,
                        out_hbm.at[cid, 0, pl.ds(sid * rows_pt, rows_pt)])

    return deg_kernel


def _make_agg_kernel(n_pad, e_pad, width):
    per_w = e_pad // NW
    n_ch = per_w // CH
    rows_pt = n_pad // NS
    n_wave = n_ch // UA
    tail = n_ch - n_wave * UA
    n_pair = (n_wave - 1) // 2
    rem = n_wave - 2 * n_pair
    mesh = plsc.VectorSubcoreMesh(core_axis_name="c", subcore_axis_name="s")

    @functools.partial(
        pl.kernel,
        out_type=jax.ShapeDtypeStruct((NC, n_pad, width), _f32),
        mesh=mesh,
        compiler_params=pltpu.CompilerParams(use_tc_tiling_on_sc=False),
        scratch_types=[
            pltpu.VMEM((n_ch, CH), jnp.int32),
            pltpu.VMEM((n_ch, CH), jnp.int32),
            pltpu.VMEM((2, UA, CH, width), _f32),
            pltpu.VMEM((rows_pt, width), _f32),
            pltpu.VMEM_SHARED((n_pad, width), _f32),
            pltpu.SemaphoreType.DMA,
            pltpu.SemaphoreType.DMA,
            pltpu.SemaphoreType.DMA,
        ],
    )
    def agg_kernel(g_hbm, ei_hbm, out_hbm,
                   sidx2, didx2, rows, stage, acc, gsem0, gsem1, ssem):
        cid = lax.axis_index("c")
        sid = lax.axis_index("s")
        wid = sid * NC + cid
        gsems = (gsem0, gsem1)

        pltpu.sync_copy(ei_hbm.at[0, pl.ds(wid * n_ch, n_ch)], sidx2)
        pltpu.sync_copy(ei_hbm.at[1, pl.ds(wid * n_ch, n_ch)], didx2)

        def _zero(i, _):
            def _zcol(j, _):
                stage[i, pl.ds(j * 16, 16)] = jnp.zeros((16,), _f32)
                return 0
            lax.fori_loop(0, width // 16, _zcol, 0)
            return 0
        lax.fori_loop(0, rows_pt, _zero, 0)
        pltpu.sync_copy(stage, acc.at[pl.ds(sid * rows_pt, rows_pt)])
        plsc.subcore_barrier()

        def _fire_g(wv, bank, count=UA):
            return [pltpu.async_copy(g_hbm.at[sidx2.at[wv * UA + b]],
                                     rows.at[bank, b], gsems[bank])
                    for b in range(count)]

        def _drain_consume(wv, bank, count=UA):
            for b in range(count):
                pltpu.make_async_copy(g_hbm.at[sidx2.at[wv * UA + b]],
                                      rows.at[bank, b], gsems[bank]).wait()
            sds = [pltpu.async_copy(rows.at[bank, b],
                                    acc.at[didx2.at[wv * UA + b]],
                                    ssem, add=True)
                   for b in range(count)]
            for d in sds:
                d.wait()

        _fire_g(0, 0)

        def _pair(i, _):
            wa = 2 * i
            _fire_g(wa + 1, 1)
            _drain_consume(wa, 0)
            _fire_g(wa + 2, 0)
            _drain_consume(wa + 1, 1)
            return 0
        lax.fori_loop(0, n_pair, _pair, 0)

        if rem == 2:
            wa = 2 * n_pair
            _fire_g(wa + 1, 1)
            _drain_consume(wa, 0)
            _drain_consume(wa + 1, 1)
        else:
            _drain_consume(2 * n_pair, 0)
        if tail:
            base = n_wave * UA
            gds = [pltpu.async_copy(g_hbm.at[sidx2.at[base + b]],
                                    rows.at[0, b], gsem0)
                   for b in range(tail)]
            for d in gds:
                d.wait()
            sds = [pltpu.async_copy(rows.at[0, b],
                                    acc.at[didx2.at[base + b]],
                                    ssem, add=True)
                   for b in range(tail)]
            for d in sds:
                d.wait()
        plsc.subcore_barrier()

        pltpu.sync_copy(acc.at[pl.ds(sid * rows_pt, rows_pt)], stage)
        pltpu.sync_copy(stage, out_hbm.at[cid, pl.ds(sid * rows_pt, rows_pt)])

    return agg_kernel


def _prep(x, w1, ccol):
    n = x.shape[0]
    hid = w1.shape[1]

    def body(x_ref, w_ref, c_ref, g_ref, sp_ref):
        deg = c_ref[...]
        dis = lax.rsqrt(deg)
        inv = 1.0 / deg
        p = jnp.dot(x_ref[...], w_ref[...], preferred_element_type=_f32)
        g_ref[...] = p * dis
        sp_ref[...] = p * inv

    return pl.pallas_call(
        body,
        out_shape=(
            jax.ShapeDtypeStruct((n, hid), _f32),
            jax.ShapeDtypeStruct((n, hid), _f32),
        ),
    )(x, w1, ccol)


def _mid(s1, sp1, b1, ccol):
    n, hid = sp1.shape

    def body(s_ref, sp_ref, bias_ref, c_ref, g2_ref, sh_ref):
        deg = c_ref[...]
        dis = lax.rsqrt(deg)
        inv = 1.0 / deg
        s = s_ref[0, :n, :] + s_ref[1, :n, :]
        h = s * dis + sp_ref[...] + bias_ref[...]
        h = jnp.maximum(h, 0.0)
        g2_ref[...] = h * dis
        sh_ref[...] = h * inv

    return pl.pallas_call(
        body,
        out_shape=(
            jax.ShapeDtypeStruct((n, hid), _f32),
            jax.ShapeDtypeStruct((n, hid), _f32),
        ),
    )(s1, sp1, b1, ccol)


def _fin(s2, sh, ccol, wmu, bmu, wls, bls):
    n = sh.shape[0]
    out_d = wmu.shape[1]

    def body(s_ref, sh_ref, c_ref, wmu_ref, bmu_ref,
             wls_ref, bls_ref, mu_ref, ls_ref):
        dis = lax.rsqrt(c_ref[...])
        s = s_ref[0, :n, :] + s_ref[1, :n, :]
        agg = s * dis + sh_ref[...]
        mu_ref[...] = jnp.dot(agg, wmu_ref[...],
                              preferred_element_type=_f32) + bmu_ref[...]
        ls_ref[...] = jnp.dot(agg, wls_ref[...],
                              preferred_element_type=_f32) + bls_ref[...]

    return pl.pallas_call(
        body,
        out_shape=(
            jax.ShapeDtypeStruct((n, out_d), _f32),
            jax.ShapeDtypeStruct((n, out_d), _f32),
        ),
    )(s2, sh, ccol, wmu, bmu, wls, bls)


def kernel(x, edge_index, W1, b1, W_mu, b_mu, W_ls, b_ls):
    n, _ = x.shape
    hid = W1.shape[1]
    out_d = W_mu.shape[1]
    e = edge_index.shape[1]

    n_pad = _round_up(n + 1, NS * 128)
    e_pad = _round_up(e, NW * CH)

    if e_pad != e:
        pad = e_pad - e
        filler = jnp.stack([jnp.zeros((pad,), jnp.int32),
                            jnp.full((pad,), n, jnp.int32)])
        ei = jnp.concatenate([edge_index, filler], axis=1)
    else:
        ei = edge_index
    ei = ei.reshape(2, e_pad // CH, CH)

    deg_k = _make_degree_kernel(n_pad, e_pad)
    agg_k = _make_agg_kernel(n_pad, e_pad, hid)

    cnt = deg_k(ei)
    ccol = (cnt[0, 0, :n] + cnt[1, 0, :n] + 1.0).reshape(n, 1)

    g1, sp1 = _prep(x, W1, ccol)
    s1 = agg_k(g1, ei)
    g2, sh = _mid(s1, sp1, b1.reshape(1, hid), ccol)
    s2 = agg_k(g2, ei)
    mu, ls = _fin(s2, sh, ccol, W_mu, b_mu.reshape(1, out_d),
                  W_ls, b_ls.reshape(1, out_d))
    return (mu, ls)
